# baseline (speedup 1.0000x reference)
import functools

import jax
import jax.numpy as jnp
from jax import lax
from jax.experimental import pallas as pl
from jax.experimental.pallas import tpu as pltpu
from jax.experimental.pallas import tpu_sc as plsc

NUM_ENTITIES = 1000000
EMBED_DIM = 64
BATCH = 16384

NC = 2
NS = 16
NW = NC * NS
B_PER_W = BATCH // NW
CHUNK = 128
NCHUNK = B_PER_W // CHUNK
NBUF = 2


def _extract(v, r):
    return lax.squeeze(lax.slice(v, (r,), (r + 1,)), (0,))


def _sc_kernel(src_hbm, tgt_hbm, emb2_hbm, rel_hbm, out_hbm,
               sidx, tidx, ridx_s, ridx_t, srows, trows, relv, outv, outs,
               sem):
    cid = lax.axis_index("c")
    sid = lax.axis_index("s")
    wid = sid * NC + cid
    base = wid * B_PER_W

    pltpu.sync_copy(rel_hbm, relv)
    for j in range(NCHUNK):
        pltpu.sync_copy(src_hbm.at[pl.ds(base + j * CHUNK, CHUNK)], sidx.at[j])
        pltpu.sync_copy(tgt_hbm.at[pl.ds(base + j * CHUNK, CHUNK)], tidx.at[j])

    for j in range(NCHUNK):
        for t in range(CHUNK // 16):
            sl = pl.ds(t * 16, 16)
            ridx_s[j, sl] = sidx[j, sl] // 2
            ridx_t[j, sl] = tidx[j, sl] // 2

    rel_q = [relv[pl.ds(q * 16, 16)] for q in range(EMBED_DIM // 16)]

    def fire(j):
        jb = j % NBUF
        return (pltpu.async_copy(emb2_hbm.at[ridx_s.at[j]], srows.at[jb], sem),
                pltpu.async_copy(emb2_hbm.at[ridx_t.at[j]], trows.at[jb], sem))

    handles = {j: fire(j) for j in range(NBUF)}

    for j in range(NCHUNK):
        h1, h2 = handles.pop(j)
        h1.wait()
        h2.wait()
        jb = j % NBUF

        def grp_body(g, _, j=j, jb=jb):
            hs16 = (sidx[j, pl.ds(g * 16, 16)] % 2) * EMBED_DIM
            ht16 = (tidx[j, pl.ds(g * 16, 16)] % 2) * EMBED_DIM
            for r in range(16):
                hs = _extract(hs16, r)
                ht = _extract(ht16, r)
                i = g * 16 + r
                acc = None
                for q in range(EMBED_DIM // 16):
                    s = srows[jb, i, pl.ds(hs + q * 16, 16)]
                    t = trows[jb, i, pl.ds(ht + q * 16, 16)]
                    d = jnp.abs(s - t + rel_q[q])
                    acc = d if acc is None else acc + d
                outs[j * CHUNK + i] = jnp.sum(acc)
            return 0

        lax.fori_loop(0, CHUNK // 16, grp_body, 0)
        if j + NBUF < NCHUNK:
            handles[j + NBUF] = fire(j + NBUF)

    lanes = lax.iota(jnp.int32, 16)

    def asm_body(g, _):
        v = jnp.zeros((16,), jnp.float32)
        for r in range(16):
            v = jnp.where(lanes == r, outs[g * 16 + r], v)
        outv[pl.ds(g * 16, 16)] = v
        return 0

    lax.fori_loop(0, B_PER_W // 16, asm_body, 0)

    pltpu.sync_copy(outv, out_hbm.at[pl.ds(base, B_PER_W)])


@jax.jit
def _transe_score(sources, targets, entity_emb, relation_emb):
    emb2 = entity_emb.reshape(NUM_ENTITIES // 2, 2 * EMBED_DIM)
    rel = relation_emb.reshape(EMBED_DIM)
    mesh = plsc.VectorSubcoreMesh(core_axis_name="c", subcore_axis_name="s")
    kern = functools.partial(
        pl.kernel,
        out_type=jax.ShapeDtypeStruct((BATCH,), jnp.float32),
        mesh=mesh,
        compiler_params=pltpu.CompilerParams(needs_layout_passes=False,
                                             use_tc_tiling_on_sc=True),
        scratch_types=[
            pltpu.VMEM((NCHUNK, CHUNK), jnp.int32),
            pltpu.VMEM((NCHUNK, CHUNK), jnp.int32),
            pltpu.VMEM((NCHUNK, CHUNK), jnp.int32),
            pltpu.VMEM((NCHUNK, CHUNK), jnp.int32),
            pltpu.VMEM((NBUF, CHUNK, 2 * EMBED_DIM), jnp.float32),
            pltpu.VMEM((NBUF, CHUNK, 2 * EMBED_DIM), jnp.float32),
            pltpu.VMEM((EMBED_DIM,), jnp.float32),
            pltpu.VMEM((B_PER_W,), jnp.float32),
            pltpu.SMEM((B_PER_W,), jnp.float32),
            pltpu.SemaphoreType.DMA,
        ],
    )(_sc_kernel)
    return kern(sources, targets, emb2, rel)


def kernel(sources, targets, entity_emb, relation_emb):
    return _transe_score(sources.astype(jnp.int32), targets.astype(jnp.int32),
                         entity_emb, relation_emb)


# speedup vs baseline: 1.5754x; 1.5754x over previous
import functools

import jax
import jax.numpy as jnp
from jax import lax
from jax.experimental import pallas as pl
from jax.experimental.pallas import tpu as pltpu
from jax.experimental.pallas import tpu_sc as plsc

NUM_ENTITIES = 1000000
EMBED_DIM = 64
BATCH = 16384

NC = 2
NS = 16
NW = NC * NS
B_PER_W = BATCH // NW
GRP = 16
NGRP = B_PER_W // GRP
NBUF = 2


def _extract(v, r):
    return lax.squeeze(lax.slice(v, (r,), (r + 1,)), (0,))


def _sc_kernel(src_hbm, tgt_hbm, emb_hbm, rel_hbm, out_hbm,
               sidx, tidx, swin, twin, relv, outv, outs, sem):
    cid = lax.axis_index("c")
    sid = lax.axis_index("s")
    wid = sid * NC + cid
    base = wid * B_PER_W

    pltpu.sync_copy(rel_hbm, relv)
    pltpu.sync_copy(src_hbm.at[pl.ds(base, B_PER_W)], sidx)
    pltpu.sync_copy(tgt_hbm.at[pl.ds(base, B_PER_W)], tidx)

    rel_q = [relv[pl.ds(q * 16, 16)] for q in range(EMBED_DIM // 16)]

    def fire(g, gb):
        sv = sidx[pl.ds(g * GRP, GRP)]
        tv = tidx[pl.ds(g * GRP, GRP)]
        for r in range(GRP):
            es = (_extract(sv, r) // 8) * 8
            et = (_extract(tv, r) // 8) * 8
            pltpu.async_copy(emb_hbm.at[pl.ds(es, 8), :], swin.at[gb, r], sem)
            pltpu.async_copy(emb_hbm.at[pl.ds(et, 8), :], twin.at[gb, r], sem)

    def drain(gb):
        for r in range(GRP):
            pltpu.make_async_copy(
                emb_hbm.at[pl.ds(0, 8), :], swin.at[gb, r], sem).wait()
            pltpu.make_async_copy(
                emb_hbm.at[pl.ds(0, 8), :], twin.at[gb, r], sem).wait()

    def compute(g, gb):
        sv = sidx[pl.ds(g * GRP, GRP)] % 8
        tv = tidx[pl.ds(g * GRP, GRP)] % 8
        for r in range(GRP):
            rs = _extract(sv, r)
            rt = _extract(tv, r)
            acc = None
            for q in range(EMBED_DIM // 16):
                s = swin[gb, r, rs, pl.ds(q * 16, 16)]
                t = twin[gb, r, rt, pl.ds(q * 16, 16)]
                d = jnp.abs(s - t + rel_q[q])
                acc = d if acc is None else acc + d
            outs[g * GRP + r] = jnp.sum(acc)

    fire(0, 0)
    fire(1, 1)

    def pair_body(k, _):
        g0 = 2 * k
        drain(0)
        compute(g0, 0)
        fire(g0 + 2, 0)
        drain(1)
        compute(g0 + 1, 1)
        fire(g0 + 3, 1)
        return 0

    lax.fori_loop(0, NGRP // 2 - 1, pair_body, 0)
    drain(0)
    compute(NGRP - 2, 0)
    drain(1)
    compute(NGRP - 1, 1)

    lanes = lax.iota(jnp.int32, 16)

    def asm_body(g, _):
        v = jnp.zeros((16,), jnp.float32)
        for r in range(16):
            v = jnp.where(lanes == r, outs[g * 16 + r], v)
        outv[pl.ds(g * 16, 16)] = v
        return 0

    lax.fori_loop(0, B_PER_W // 16, asm_body, 0)

    pltpu.sync_copy(outv, out_hbm.at[pl.ds(base, B_PER_W)])


@jax.jit
def _transe_score(sources, targets, entity_emb, relation_emb):
    rel = relation_emb.reshape(EMBED_DIM)
    mesh = plsc.VectorSubcoreMesh(core_axis_name="c", subcore_axis_name="s")
    kern = functools.partial(
        pl.kernel,
        out_type=jax.ShapeDtypeStruct((BATCH,), jnp.float32),
        mesh=mesh,
        compiler_params=pltpu.CompilerParams(needs_layout_passes=False,
                                             use_tc_tiling_on_sc=True),
        scratch_types=[
            pltpu.VMEM((B_PER_W,), jnp.int32),
            pltpu.VMEM((B_PER_W,), jnp.int32),
            pltpu.VMEM((NBUF, GRP, 8, EMBED_DIM), jnp.float32),
            pltpu.VMEM((NBUF, GRP, 8, EMBED_DIM), jnp.float32),
            pltpu.VMEM((EMBED_DIM,), jnp.float32),
            pltpu.VMEM((B_PER_W,), jnp.float32),
            pltpu.SMEM((B_PER_W,), jnp.float32),
            pltpu.SemaphoreType.DMA,
        ],
    )(_sc_kernel)
    return kern(sources, targets, entity_emb, rel)


def kernel(sources, targets, entity_emb, relation_emb):
    return _transe_score(sources.astype(jnp.int32), targets.astype(jnp.int32),
                         entity_emb, relation_emb)
